# Initial kernel scaffold; baseline (speedup 1.0000x reference)
import jax
import jax.numpy as jnp
from jax.experimental import pallas as pl


def kernel(pose_features, pos_emb_table, rel_table):
    raise NotImplementedError("write your pallas kernel here")


# speedup vs baseline: 14.3403x; 14.3403x over previous
import functools
import jax
import jax.numpy as jnp
from jax.experimental import pallas as pl
from jax.experimental.pallas import tpu as pltpu

S = 512
H = 128
R = 8


def _body(pose_ref, pos_ref, relrev_ref, out_ref, emb_ref, mrev_ref):
    p = pl.program_id(0)

    @pl.when(p == 0)
    def _init():
        emb_ref[...] = pose_ref[0] + pos_ref[...]
        mrev_ref[...] = jnp.mean(relrev_ref[...], axis=1, keepdims=True)

    i0 = p * R
    rows = []
    for r in range(R):
        start = (S - 1) - (i0 + r)
        rows.append(mrev_ref[pl.ds(start, S), :])
    f = jnp.stack(rows, axis=0)
    out_ref[0] = emb_ref[...][None, :, :] * (1.0 + 0.1 * f)


def kernel(pose_features, pos_emb_table, rel_table):
    relrev = jnp.concatenate(
        [jnp.flip(rel_table, axis=0), jnp.zeros((1, H), jnp.float32)], axis=0
    )
    grid = S // R
    out = pl.pallas_call(
        _body,
        grid=(grid,),
        in_specs=[
            pl.BlockSpec((1, S, H), lambda p: (0, 0, 0)),
            pl.BlockSpec((S, H), lambda p: (0, 0)),
            pl.BlockSpec((1024, H), lambda p: (0, 0)),
        ],
        out_specs=pl.BlockSpec((1, R, S, H), lambda p: (0, p, 0, 0)),
        out_shape=jax.ShapeDtypeStruct((1, S, S, H), jnp.float32),
        scratch_shapes=[
            pltpu.VMEM((S, H), jnp.float32),
            pltpu.VMEM((1024, 1), jnp.float32),
        ],
    )(pose_features, pos_emb_table, relrev)
    return out


# speedup vs baseline: 17.0889x; 1.1917x over previous
import functools
import jax
import jax.numpy as jnp
from jax.experimental import pallas as pl
from jax.experimental.pallas import tpu as pltpu

S = 512
H = 128
R = 16


def _body(pose_ref, pos_ref, relrev_ref, out_ref, emb_ref, mrev_ref):
    p = pl.program_id(0)

    @pl.when(p == 0)
    def _init():
        emb_ref[...] = pose_ref[0] + pos_ref[...]
        mrev_ref[...] = jnp.mean(relrev_ref[...], axis=1, keepdims=True)

    i0 = p * R
    rows = []
    for r in range(R):
        start = (S - 1) - (i0 + r)
        rows.append(mrev_ref[pl.ds(start, S), :])
    f = jnp.stack(rows, axis=0)
    out_ref[0] = emb_ref[...][None, :, :] * (1.0 + 0.1 * f)


def kernel(pose_features, pos_emb_table, rel_table):
    relrev = jnp.concatenate(
        [jnp.flip(rel_table, axis=0), jnp.zeros((1, H), jnp.float32)], axis=0
    )
    grid = S // R
    out = pl.pallas_call(
        _body,
        grid=(grid,),
        in_specs=[
            pl.BlockSpec((1, S, H), lambda p: (0, 0, 0)),
            pl.BlockSpec((S, H), lambda p: (0, 0)),
            pl.BlockSpec((1024, H), lambda p: (0, 0)),
        ],
        out_specs=pl.BlockSpec((1, R, S, H), lambda p: (0, p, 0, 0)),
        out_shape=jax.ShapeDtypeStruct((1, S, S, H), jnp.float32),
        scratch_shapes=[
            pltpu.VMEM((S, H), jnp.float32),
            pltpu.VMEM((1024, 1), jnp.float32),
        ],
    )(pose_features, pos_emb_table, relrev)
    return out


# speedup vs baseline: 18.6187x; 1.0895x over previous
import functools
import jax
import jax.numpy as jnp
from jax.experimental import pallas as pl
from jax.experimental.pallas import tpu as pltpu

S = 512
H = 128
R = 32


def _body(pose_ref, pos_ref, relrev_ref, out_ref, emb_ref, mrev_ref):
    p = pl.program_id(0)

    @pl.when(p == 0)
    def _init():
        emb_ref[...] = pose_ref[0] + pos_ref[...]
        mrev_ref[...] = jnp.mean(relrev_ref[...], axis=1, keepdims=True)

    i0 = p * R
    rows = []
    for r in range(R):
        start = (S - 1) - (i0 + r)
        rows.append(mrev_ref[pl.ds(start, S), :])
    f = jnp.stack(rows, axis=0)
    out_ref[0] = emb_ref[...][None, :, :] * (1.0 + 0.1 * f)


def kernel(pose_features, pos_emb_table, rel_table):
    relrev = jnp.concatenate(
        [jnp.flip(rel_table, axis=0), jnp.zeros((1, H), jnp.float32)], axis=0
    )
    grid = S // R
    out = pl.pallas_call(
        _body,
        grid=(grid,),
        in_specs=[
            pl.BlockSpec((1, S, H), lambda p: (0, 0, 0)),
            pl.BlockSpec((S, H), lambda p: (0, 0)),
            pl.BlockSpec((1024, H), lambda p: (0, 0)),
        ],
        out_specs=pl.BlockSpec((1, R, S, H), lambda p: (0, p, 0, 0)),
        out_shape=jax.ShapeDtypeStruct((1, S, S, H), jnp.float32),
        scratch_shapes=[
            pltpu.VMEM((S, H), jnp.float32),
            pltpu.VMEM((1024, 1), jnp.float32),
        ],
    )(pose_features, pos_emb_table, relrev)
    return out


# speedup vs baseline: 18.6597x; 1.0022x over previous
import functools
import jax
import jax.numpy as jnp
from jax.experimental import pallas as pl
from jax.experimental.pallas import tpu as pltpu

S = 512
H = 128
R = 64


def _body(pose_ref, pos_ref, relrev_ref, out_ref, emb_ref, mrev_ref):
    p = pl.program_id(0)

    @pl.when(p == 0)
    def _init():
        emb_ref[...] = pose_ref[0] + pos_ref[...]
        mrev_ref[...] = jnp.mean(relrev_ref[...], axis=1, keepdims=True)

    i0 = p * R
    rows = []
    for r in range(R):
        start = (S - 1) - (i0 + r)
        rows.append(mrev_ref[pl.ds(start, S), :])
    f = jnp.stack(rows, axis=0)
    out_ref[0] = emb_ref[...][None, :, :] * (1.0 + 0.1 * f)


def kernel(pose_features, pos_emb_table, rel_table):
    relrev = jnp.concatenate(
        [jnp.flip(rel_table, axis=0), jnp.zeros((1, H), jnp.float32)], axis=0
    )
    grid = S // R
    out = pl.pallas_call(
        _body,
        grid=(grid,),
        in_specs=[
            pl.BlockSpec((1, S, H), lambda p: (0, 0, 0)),
            pl.BlockSpec((S, H), lambda p: (0, 0)),
            pl.BlockSpec((1024, H), lambda p: (0, 0)),
        ],
        out_specs=pl.BlockSpec((1, R, S, H), lambda p: (0, p, 0, 0)),
        out_shape=jax.ShapeDtypeStruct((1, S, S, H), jnp.float32),
        scratch_shapes=[
            pltpu.VMEM((S, H), jnp.float32),
            pltpu.VMEM((1024, 1), jnp.float32),
        ],
    )(pose_features, pos_emb_table, relrev)
    return out


# speedup vs baseline: 19.9695x; 1.0702x over previous
import functools
import jax
import jax.numpy as jnp
from jax.experimental import pallas as pl
from jax.experimental.pallas import tpu as pltpu

S = 512
H = 128
R = 64


def _body(pose_ref, pos_ref, relrev_ref, out_ref, emb_ref, grep_ref):
    p = pl.program_id(0)

    @pl.when(p == 0)
    def _init():
        emb_ref[...] = pose_ref[0] + pos_ref[...]
        m = jnp.mean(relrev_ref[...], axis=1, keepdims=True)
        grep_ref[...] = jnp.broadcast_to(1.0 + 0.1 * m, (1024, H))

    i0 = p * R
    emb = emb_ref[...]
    for r in range(R):
        start = (S - 1) - (i0 + r)
        out_ref[0, r] = emb * grep_ref[pl.ds(start, S), :]


def kernel(pose_features, pos_emb_table, rel_table):
    relrev = jnp.concatenate(
        [jnp.flip(rel_table, axis=0), jnp.zeros((1, H), jnp.float32)], axis=0
    )
    grid = S // R
    out = pl.pallas_call(
        _body,
        grid=(grid,),
        in_specs=[
            pl.BlockSpec((1, S, H), lambda p: (0, 0, 0)),
            pl.BlockSpec((S, H), lambda p: (0, 0)),
            pl.BlockSpec((1024, H), lambda p: (0, 0)),
        ],
        out_specs=pl.BlockSpec((1, R, S, H), lambda p: (0, p, 0, 0)),
        out_shape=jax.ShapeDtypeStruct((1, S, S, H), jnp.float32),
        scratch_shapes=[
            pltpu.VMEM((S, H), jnp.float32),
            pltpu.VMEM((1024, H), jnp.float32),
        ],
    )(pose_features, pos_emb_table, relrev)
    return out


# speedup vs baseline: 20.7187x; 1.0375x over previous
import functools
import jax
import jax.numpy as jnp
from jax.experimental import pallas as pl
from jax.experimental.pallas import tpu as pltpu

S = 512
H = 128
R = 32


def _body(pose_ref, pos_ref, relrev_ref, out_ref, emb_ref, grep_ref):
    p = pl.program_id(0)

    @pl.when(p == 0)
    def _init():
        emb_ref[...] = pose_ref[0] + pos_ref[...]
        m = jnp.mean(relrev_ref[...], axis=1, keepdims=True)
        grep_ref[...] = jnp.broadcast_to(1.0 + 0.1 * m, (1024, H))

    i0 = p * R
    emb = emb_ref[...]
    for r in range(R):
        start = (S - 1) - (i0 + r)
        out_ref[0, r] = emb * grep_ref[pl.ds(start, S), :]


def kernel(pose_features, pos_emb_table, rel_table):
    relrev = jnp.concatenate(
        [jnp.flip(rel_table, axis=0), jnp.zeros((1, H), jnp.float32)], axis=0
    )
    grid = S // R
    out = pl.pallas_call(
        _body,
        grid=(grid,),
        in_specs=[
            pl.BlockSpec((1, S, H), lambda p: (0, 0, 0)),
            pl.BlockSpec((S, H), lambda p: (0, 0)),
            pl.BlockSpec((1024, H), lambda p: (0, 0)),
        ],
        out_specs=pl.BlockSpec((1, R, S, H), lambda p: (0, p, 0, 0)),
        out_shape=jax.ShapeDtypeStruct((1, S, S, H), jnp.float32),
        scratch_shapes=[
            pltpu.VMEM((S, H), jnp.float32),
            pltpu.VMEM((1024, H), jnp.float32),
        ],
    )(pose_features, pos_emb_table, relrev)
    return out


# speedup vs baseline: 21.1537x; 1.0210x over previous
import functools
import jax
import jax.numpy as jnp
from jax.experimental import pallas as pl
from jax.experimental.pallas import tpu as pltpu

S = 512
H = 128
R = 16


def _body(pose_ref, pos_ref, relrev_ref, out_ref, emb_ref, grep_ref):
    p = pl.program_id(0)

    @pl.when(p == 0)
    def _init():
        emb_ref[...] = pose_ref[0] + pos_ref[...]
        m = jnp.mean(relrev_ref[...], axis=1, keepdims=True)
        grep_ref[...] = jnp.broadcast_to(1.0 + 0.1 * m, (1024, H))

    i0 = p * R
    emb = emb_ref[...]
    for r in range(R):
        start = (S - 1) - (i0 + r)
        out_ref[0, r] = emb * grep_ref[pl.ds(start, S), :]


def kernel(pose_features, pos_emb_table, rel_table):
    relrev = jnp.concatenate(
        [jnp.flip(rel_table, axis=0), jnp.zeros((1, H), jnp.float32)], axis=0
    )
    grid = S // R
    out = pl.pallas_call(
        _body,
        grid=(grid,),
        in_specs=[
            pl.BlockSpec((1, S, H), lambda p: (0, 0, 0)),
            pl.BlockSpec((S, H), lambda p: (0, 0)),
            pl.BlockSpec((1024, H), lambda p: (0, 0)),
        ],
        out_specs=pl.BlockSpec((1, R, S, H), lambda p: (0, p, 0, 0)),
        out_shape=jax.ShapeDtypeStruct((1, S, S, H), jnp.float32),
        scratch_shapes=[
            pltpu.VMEM((S, H), jnp.float32),
            pltpu.VMEM((1024, H), jnp.float32),
        ],
    )(pose_features, pos_emb_table, relrev)
    return out
